# baseline (speedup 1.0000x reference)
import functools

import jax
import jax.numpy as jnp
from jax import lax
from jax.experimental import pallas as pl
from jax.experimental.pallas import tpu as pltpu
from jax.experimental.pallas import tpu_sc as plsc

LATENT = 16
LANES = 16


def _gmf_sc(user, item, user_table, item_table):
    B = user.shape[0]
    info = plsc.get_sparse_core_info()
    NC, NS = info.num_cores, info.num_subcores
    NW = NC * NS
    assert B % (8 * NW) == 0
    b_per_w = B // NW
    n_groups = b_per_w // LANES

    mesh = plsc.VectorSubcoreMesh(core_axis_name="c", subcore_axis_name="s")

    @functools.partial(
        pl.kernel,
        mesh=mesh,
        compiler_params=pltpu.CompilerParams(
            needs_layout_passes=False, use_tc_tiling_on_sc=False),
        out_type=jax.ShapeDtypeStruct((B,), jnp.float32),
        scratch_types=[
            pltpu.VMEM((b_per_w,), jnp.int32),
            pltpu.VMEM((b_per_w,), jnp.int32),
            pltpu.VMEM((b_per_w, LATENT), jnp.float32),
            pltpu.VMEM((b_per_w, LATENT), jnp.float32),
            pltpu.VMEM((b_per_w,), jnp.float32),
            pltpu.SemaphoreType.DMA,
        ],
    )
    def gmf_kernel(user_hbm, item_hbm, ut_hbm, it_hbm, out_hbm,
                   uidx_v, iidx_v, urows_v, irows_v, out_v, sem):
        wid = lax.axis_index("s") * NC + lax.axis_index("c")
        base = wid * b_per_w
        pltpu.sync_copy(user_hbm.at[pl.ds(base, b_per_w)], uidx_v)
        pltpu.sync_copy(item_hbm.at[pl.ds(base, b_per_w)], iidx_v)
        cp_u = pltpu.async_copy(ut_hbm.at[uidx_v], urows_v, sem)
        cp_i = pltpu.async_copy(it_hbm.at[iidx_v], irows_v, sem)
        cp_u.wait()
        cp_i.wait()

        lane = lax.iota(jnp.int32, LANES)

        def body(j, carry):
            row_ids = lane + j * LANES
            acc = jnp.zeros((LANES,), jnp.float32)
            for d in range(LATENT):
                col = jnp.full((LANES,), d, jnp.int32)
                uc = plsc.load_gather(urows_v, [row_ids, col])
                ic = plsc.load_gather(irows_v, [row_ids, col])
                acc = acc + uc * ic
            z = jnp.exp(-jnp.abs(acc))
            r = jnp.where(acc >= 0.0, 1.0 / (1.0 + z), z / (1.0 + z))
            out_v[pl.ds(j * LANES, LANES)] = r
            return carry

        lax.fori_loop(0, n_groups, body, 0)
        pltpu.sync_copy(out_v, out_hbm.at[pl.ds(base, b_per_w)])

    return gmf_kernel(user, item, user_table, item_table)


def kernel(user, item, user_table, item_table):
    return _gmf_sc(user, item, user_table, item_table)
